# baseline (speedup 1.0000x reference)
import dataclasses

import jax
import jax.numpy as jnp
from jax import lax
from jax.experimental import pallas as pl
from jax.experimental.pallas import tpu as pltpu
from jax.experimental.pallas import tpu_sc as plsc

M_ROWS = 1000000
DIM = 64
BATCH = 16384
MOM = 0.5

NC = 2
NS = 16
NW = NC * NS
L = 16
QUARTERS = DIM // L

BW = 128
OWN_W = 31232
NBLK = OWN_W // BW
NBLK_LAST = NBLK + 4
TAIL_W = M_ROWS - (NW - 1) * OWN_W - NBLK_LAST * BW
AUX_W = NBLK_LAST * BW + TAIL_W + L * 2
XS = BATCH // NS
RND = 2048
YB = 256


def _update_body(x_hbm, y_hbm, mt_hbm, mtail_hbm, out_hbm, otail_hbm,
                 ybuf_v, rows_c, pos_c, aux,
                 blk_a, blk_b, blk_t, wrk_col_a, wrk_pos_a, wrk_col_b,
                 wrk_pos_b, xbuf_a, xbuf_b,
                 sem_xa, sem_xb, sem_ia, sem_ib, sem_oa, sem_ob):
    cid = lax.axis_index("c")
    sid = lax.axis_index("s")
    wid = sid * NC + cid
    lo = wid * OWN_W
    hi = jnp.where(wid == NW - 1, jnp.int32(M_ROWS), lo + OWN_W)
    nblk = jnp.where(wid == NW - 1, jnp.int32(NBLK_LAST), jnp.int32(NBLK))
    lane = lax.iota(jnp.int32, L)

    pltpu.async_copy(y_hbm, ybuf_v, sem_ia).wait()

    neg1 = jnp.full((L,), -1, dtype=jnp.int32)

    @pl.loop(0, AUX_W // L)
    def _init(k):
        aux[pl.ds(k * L, L)] = neg1

    @pl.loop(0, BATCH // RND)
    def _round(r):
        base_r = r * RND

        def p1_body(cb, nacc):
            for t in range(YB // L):
                v = ybuf_v[pl.ds(base_r + cb * YB + t * L, L)]
                own = (v >= lo) & (v < hi)
                posv = lane + base_r + cb * YB + t * L
                plsc.store_compressed(rows_c.at[pl.ds(nacc, L)], v, mask=own)
                plsc.store_compressed(pos_c.at[pl.ds(nacc, L)], posv,
                                      mask=own)
                nacc = nacc + jnp.sum(own.astype(jnp.int32))
            return nacc

        n = lax.fori_loop(0, RND // YB, p1_body, jnp.int32(0))
        nblk2 = lax.div(n + (L - 1), jnp.int32(L))

        def p2_body(i, carry):
            base = i * L
            v = rows_c[pl.ds(base, L)] - lo
            p = pos_c[pl.ds(base, L)]
            valid = (lane + base) < n
            for j in range(L):
                plsc.store_scatter(aux, [v], p, mask=valid & (lane == j))
            return carry

        lax.fori_loop(0, nblk2, p2_body, jnp.int32(0))


    def make_copy(c0, blk, wrk_col, wrk_pos, xbuf, sem_i, sem_o, sem_x,
                  width, src=None, dst=None):
        def start_in():
            if src is not None:
                return pltpu.async_copy(src, blk, sem_i)
            return pltpu.async_copy(mt_hbm.at[:, pl.ds(c0, width)], blk,
                                    sem_i)

        def scan():
            def scan_piece(t, cnt):
                wv = aux[pl.ds(c0 - lo + t * L, L)]
                keep = wv >= jnp.int32(0)
                plsc.store_compressed(wrk_col.at[pl.ds(cnt, L)],
                                      lane + t * L, mask=keep)
                plsc.store_compressed(wrk_pos.at[pl.ds(cnt, L)], wv,
                                      mask=keep)
                return cnt + jnp.sum(keep.astype(jnp.int32))

            cnt = jnp.int32(0)
            for t in range(width // L):
                cnt = scan_piece(t, cnt)

            @pl.loop(0, cnt)
            def _prefetch(u):
                pj = wrk_pos[pl.ds(u, L)][0]
                pltpu.async_copy(x_hbm.at[pj], xbuf.at[u], sem_x)

            return cnt

        def process(cnt):
            @pl.loop(0, cnt)
            def _drain(u):
                pj = wrk_pos[pl.ds(u, L)][0]
                pltpu.make_async_copy(x_hbm.at[pj], xbuf.at[u], sem_x).wait()

            @pl.loop(0, cnt)
            def _upd(u):
                cj = wrk_col[pl.ds(u, L)][0]
                colc = jnp.full((L,), cj, dtype=jnp.int32)
                ws = [
                    plsc.load_gather(blk, [lane + q * L, colc]) * MOM
                    + xbuf[u, pl.ds(q * L, L)] * (1.0 - MOM)
                    for q in range(QUARTERS)
                ]
                s = jnp.float32(0.0)
                for wq in ws:
                    s = s + jnp.sum(wq * wq)
                s = jnp.maximum(s, jnp.float32(1e-24))
                sv = jnp.full((L,), s, dtype=jnp.float32)
                bits = lax.bitcast_convert_type(sv, jnp.int32)
                yv = lax.bitcast_convert_type(
                    jnp.int32(0x5F3759DF) - lax.shift_right_logical(bits, 1),
                    jnp.float32,
                )
                for _ in range(3):
                    yv = yv * (1.5 - 0.5 * sv * yv * yv)
                for q, wq in enumerate(ws):
                    plsc.store_scatter(blk, [lane + q * L, colc], wq * yv)

        def start_out():
            if dst is not None:
                return pltpu.async_copy(blk, dst, sem_o)
            return pltpu.async_copy(blk, out_hbm.at[:, pl.ds(c0, width)],
                                    sem_o)

        return start_in, scan, process, start_out

    def block_c0(k):
        return pl.multiple_of(lo + k * BW, BW)

    def pair_body(k):
        in_a, scan_a, proc_a, out_a = make_copy(
            block_c0(k), blk_a, wrk_col_a, wrk_pos_a, xbuf_a,
            sem_ia, sem_oa, sem_xa, BW)
        in_b, scan_b, proc_b, out_b = make_copy(
            block_c0(k + 1), blk_b, wrk_col_b, wrk_pos_b, xbuf_b,
            sem_ib, sem_ob, sem_xb, BW)
        @pl.when(k >= 2)
        def _():
            pltpu.make_async_copy(
                blk_a, out_hbm.at[:, pl.ds(block_c0(k - 2), BW)],
                sem_oa).wait()
            pltpu.make_async_copy(
                blk_b, out_hbm.at[:, pl.ds(block_c0(k - 1), BW)],
                sem_ob).wait()

        ca = in_a()
        cb = in_b()
        cnt_a = scan_a()
        cnt_b = scan_b()
        ca.wait()
        proc_a(cnt_a)
        out_a()
        cb.wait()
        proc_b(cnt_b)
        out_b()

    pl.loop(0, nblk, step=2)(pair_body)

    pltpu.make_async_copy(
        blk_a, out_hbm.at[:, pl.ds(block_c0(nblk - 2), BW)], sem_oa).wait()
    pltpu.make_async_copy(
        blk_b, out_hbm.at[:, pl.ds(block_c0(nblk - 1), BW)], sem_ob).wait()

    @pl.when(wid == NW - 1)
    def _tail():
        c0t = lo + NBLK_LAST * BW
        t_in, t_scan, t_proc, t_out = make_copy(
            c0t, blk_t, wrk_col_a, wrk_pos_a, xbuf_a,
            sem_ia, sem_oa, sem_xa, TAIL_W, src=mtail_hbm, dst=otail_hbm)
        ct = t_in()
        cnt_t = t_scan()
        ct.wait()
        t_proc(cnt_t)
        t_out().wait()


_cp = pltpu.CompilerParams()
if "needs_layout_passes" in pltpu.CompilerParams.__dataclass_fields__:
    _cp = dataclasses.replace(_cp, needs_layout_passes=False)

_update = pl.kernel(
    _update_body,
    out_type=(jax.ShapeDtypeStruct((DIM, M_ROWS), jnp.float32),
              jax.ShapeDtypeStruct((DIM, TAIL_W), jnp.float32)),
    mesh=plsc.VectorSubcoreMesh(
        core_axis_name="c", subcore_axis_name="s", num_cores=NC, num_subcores=NS
    ),
    compiler_params=_cp,
    scratch_types=[
        pltpu.VMEM((BATCH,), jnp.int32),
        pltpu.VMEM((RND + L,), jnp.int32),
        pltpu.VMEM((RND + L,), jnp.int32),
        pltpu.VMEM((AUX_W,), jnp.int32),
        pltpu.VMEM((DIM, BW), jnp.float32),
        pltpu.VMEM((DIM, BW), jnp.float32),
        pltpu.VMEM((DIM, TAIL_W), jnp.float32),
        pltpu.VMEM((BW + L,), jnp.int32),
        pltpu.VMEM((BW + L,), jnp.int32),
        pltpu.VMEM((BW + L,), jnp.int32),
        pltpu.VMEM((BW + L,), jnp.int32),
        pltpu.VMEM((BW, DIM), jnp.float32),
        pltpu.VMEM((BW, DIM), jnp.float32),
        pltpu.SemaphoreType.DMA,
        pltpu.SemaphoreType.DMA,
        pltpu.SemaphoreType.DMA,
        pltpu.SemaphoreType.DMA,
        pltpu.SemaphoreType.DMA,
        pltpu.SemaphoreType.DMA,
    ],
)


TAIL0 = (NW - 1) * OWN_W + NBLK_LAST * BW


def kernel(memory, x, y):
    mt = memory.T
    mtail = lax.slice(mt, (0, TAIL0), (DIM, M_ROWS))
    out_t, otail = _update(x, y, mt, mtail)
    out_t = lax.dynamic_update_slice(out_t, otail, (0, TAIL0))
    return out_t.T


# speedup vs baseline: 1.1432x; 1.1432x over previous
import dataclasses

import jax
import jax.numpy as jnp
from jax import lax
from jax.experimental import pallas as pl
from jax.experimental.pallas import tpu as pltpu
from jax.experimental.pallas import tpu_sc as plsc

M_ROWS = 1000000
DIM = 64
BATCH = 16384
MOM = 0.5

NC = 2
NS = 16
NW = NC * NS
L = 16
QUARTERS = DIM // L

BW = 256
OWN_W = 31232
NBLK = OWN_W // BW
NBLK_LAST = NBLK + 2
TAIL_W = M_ROWS - (NW - 1) * OWN_W - NBLK_LAST * BW
AUX_W = NBLK_LAST * BW + TAIL_W + L * 2
XS = BATCH // NS
RND = 2048
YB = 256
XCAP = 64


def _update_body(x_hbm, y_hbm, mt_hbm, mtail_hbm, out_hbm, otail_hbm,
                 ybuf_v, rows_c, pos_c, aux,
                 blk_a, blk_b, blk_t, wrk_col_a, wrk_pos_a, wrk_col_b,
                 wrk_pos_b, xbuf_a, xbuf_b,
                 sem_xa, sem_xb, sem_ia, sem_ib, sem_oa, sem_ob):
    cid = lax.axis_index("c")
    sid = lax.axis_index("s")
    wid = sid * NC + cid
    lo = wid * OWN_W
    hi = jnp.where(wid == NW - 1, jnp.int32(M_ROWS), lo + OWN_W)
    nblk = jnp.where(wid == NW - 1, jnp.int32(NBLK_LAST), jnp.int32(NBLK))
    lane = lax.iota(jnp.int32, L)

    pltpu.async_copy(y_hbm, ybuf_v, sem_ia).wait()

    neg1 = jnp.full((L,), -1, dtype=jnp.int32)

    @pl.loop(0, AUX_W // L)
    def _init(k):
        aux[pl.ds(k * L, L)] = neg1

    @pl.loop(0, BATCH // RND)
    def _round(r):
        base_r = r * RND

        def p1_body(cb, nacc):
            for t in range(YB // L):
                v = ybuf_v[pl.ds(base_r + cb * YB + t * L, L)]
                own = (v >= lo) & (v < hi)
                posv = lane + base_r + cb * YB + t * L
                plsc.store_compressed(rows_c.at[pl.ds(nacc, L)], v, mask=own)
                plsc.store_compressed(pos_c.at[pl.ds(nacc, L)], posv,
                                      mask=own)
                nacc = nacc + jnp.sum(own.astype(jnp.int32))
            return nacc

        n = lax.fori_loop(0, RND // YB, p1_body, jnp.int32(0))
        nblk2 = lax.div(n + (L - 1), jnp.int32(L))

        def p2_body(i, carry):
            base = i * L
            v = rows_c[pl.ds(base, L)] - lo
            p = pos_c[pl.ds(base, L)]
            valid = (lane + base) < n
            for j in range(L):
                plsc.store_scatter(aux, [v], p, mask=valid & (lane == j))
            return carry

        lax.fori_loop(0, nblk2, p2_body, jnp.int32(0))


    def make_copy(c0, blk, wrk_col, wrk_pos, xbuf, sem_i, sem_o, sem_x,
                  width, src=None, dst=None):
        def start_in():
            if src is not None:
                return pltpu.async_copy(src, blk, sem_i)
            return pltpu.async_copy(mt_hbm.at[:, pl.ds(c0, width)], blk,
                                    sem_i)

        def scan():
            def scan_piece(t, cnt):
                wv = aux[pl.ds(c0 - lo + t * L, L)]
                keep = wv >= jnp.int32(0)
                plsc.store_compressed(wrk_col.at[pl.ds(cnt, L)],
                                      lane + t * L, mask=keep)
                plsc.store_compressed(wrk_pos.at[pl.ds(cnt, L)], wv,
                                      mask=keep)
                return cnt + jnp.sum(keep.astype(jnp.int32))

            cnt = jnp.int32(0)
            for t in range(width // L):
                cnt = scan_piece(t, cnt)

            @pl.loop(0, jnp.minimum(cnt, XCAP))
            def _prefetch(u):
                pj = wrk_pos[pl.ds(u, L)][0]
                pltpu.async_copy(x_hbm.at[pj], xbuf.at[u], sem_x)

            return cnt

        def process(cnt):
            def chunk(w0):
                csz = jnp.minimum(cnt - w0, XCAP)

                @pl.when(w0 > 0)
                def _():
                    @pl.loop(0, csz)
                    def _prefetch(uu):
                        pj = wrk_pos[pl.ds(w0 + uu, L)][0]
                        pltpu.async_copy(x_hbm.at[pj], xbuf.at[uu], sem_x)

                @pl.loop(0, csz)
                def _drain(uu):
                    pj = wrk_pos[pl.ds(w0 + uu, L)][0]
                    pltpu.make_async_copy(x_hbm.at[pj], xbuf.at[uu],
                                          sem_x).wait()

                @pl.loop(0, csz)
                def _upd(uu):
                    u = w0 + uu
                    cj = wrk_col[pl.ds(u, L)][0]
                    colc = jnp.full((L,), cj, dtype=jnp.int32)
                    ws = [
                        plsc.load_gather(blk, [lane + q * L, colc]) * MOM
                        + xbuf[uu, pl.ds(q * L, L)] * (1.0 - MOM)
                        for q in range(QUARTERS)
                    ]
                    s = jnp.float32(0.0)
                    for wq in ws:
                        s = s + jnp.sum(wq * wq)
                    s = jnp.maximum(s, jnp.float32(1e-24))
                    sv = jnp.full((L,), s, dtype=jnp.float32)
                    bits = lax.bitcast_convert_type(sv, jnp.int32)
                    yv = lax.bitcast_convert_type(
                        jnp.int32(0x5F3759DF)
                        - lax.shift_right_logical(bits, 1),
                        jnp.float32,
                    )
                    for _ in range(3):
                        yv = yv * (1.5 - 0.5 * sv * yv * yv)
                    for q, wq in enumerate(ws):
                        plsc.store_scatter(blk, [lane + q * L, colc],
                                           wq * yv)

            pl.loop(0, cnt, step=XCAP)(chunk)

        def start_out():
            if dst is not None:
                return pltpu.async_copy(blk, dst, sem_o)
            return pltpu.async_copy(blk, out_hbm.at[:, pl.ds(c0, width)],
                                    sem_o)

        return start_in, scan, process, start_out

    def block_c0(k):
        return pl.multiple_of(lo + k * BW, BW)

    def pair_body(k):
        in_a, scan_a, proc_a, out_a = make_copy(
            block_c0(k), blk_a, wrk_col_a, wrk_pos_a, xbuf_a,
            sem_ia, sem_oa, sem_xa, BW)
        in_b, scan_b, proc_b, out_b = make_copy(
            block_c0(k + 1), blk_b, wrk_col_b, wrk_pos_b, xbuf_b,
            sem_ib, sem_ob, sem_xb, BW)
        @pl.when(k >= 2)
        def _():
            pltpu.make_async_copy(
                blk_a, out_hbm.at[:, pl.ds(block_c0(k - 2), BW)],
                sem_oa).wait()
            pltpu.make_async_copy(
                blk_b, out_hbm.at[:, pl.ds(block_c0(k - 1), BW)],
                sem_ob).wait()

        ca = in_a()
        cb = in_b()
        cnt_a = scan_a()
        cnt_b = scan_b()
        ca.wait()
        proc_a(cnt_a)
        out_a()
        cb.wait()
        proc_b(cnt_b)
        out_b()

    pl.loop(0, nblk, step=2)(pair_body)

    pltpu.make_async_copy(
        blk_a, out_hbm.at[:, pl.ds(block_c0(nblk - 2), BW)], sem_oa).wait()
    pltpu.make_async_copy(
        blk_b, out_hbm.at[:, pl.ds(block_c0(nblk - 1), BW)], sem_ob).wait()

    @pl.when(wid == NW - 1)
    def _tail():
        c0t = lo + NBLK_LAST * BW
        t_in, t_scan, t_proc, t_out = make_copy(
            c0t, blk_t, wrk_col_a, wrk_pos_a, xbuf_a,
            sem_ia, sem_oa, sem_xa, TAIL_W, src=mtail_hbm, dst=otail_hbm)
        ct = t_in()
        cnt_t = t_scan()
        ct.wait()
        t_proc(cnt_t)
        t_out().wait()


_cp = pltpu.CompilerParams()
if "needs_layout_passes" in pltpu.CompilerParams.__dataclass_fields__:
    _cp = dataclasses.replace(_cp, needs_layout_passes=False)

_update = pl.kernel(
    _update_body,
    out_type=(jax.ShapeDtypeStruct((DIM, M_ROWS), jnp.float32),
              jax.ShapeDtypeStruct((DIM, TAIL_W), jnp.float32)),
    mesh=plsc.VectorSubcoreMesh(
        core_axis_name="c", subcore_axis_name="s", num_cores=NC, num_subcores=NS
    ),
    compiler_params=_cp,
    scratch_types=[
        pltpu.VMEM((BATCH,), jnp.int32),
        pltpu.VMEM((RND + L,), jnp.int32),
        pltpu.VMEM((RND + L,), jnp.int32),
        pltpu.VMEM((AUX_W,), jnp.int32),
        pltpu.VMEM((DIM, BW), jnp.float32),
        pltpu.VMEM((DIM, BW), jnp.float32),
        pltpu.VMEM((DIM, TAIL_W), jnp.float32),
        pltpu.VMEM((BW + L,), jnp.int32),
        pltpu.VMEM((BW + L,), jnp.int32),
        pltpu.VMEM((BW + L,), jnp.int32),
        pltpu.VMEM((BW + L,), jnp.int32),
        pltpu.VMEM((XCAP, DIM), jnp.float32),
        pltpu.VMEM((XCAP, DIM), jnp.float32),
        pltpu.SemaphoreType.DMA,
        pltpu.SemaphoreType.DMA,
        pltpu.SemaphoreType.DMA,
        pltpu.SemaphoreType.DMA,
        pltpu.SemaphoreType.DMA,
        pltpu.SemaphoreType.DMA,
    ],
)


TAIL0 = (NW - 1) * OWN_W + NBLK_LAST * BW


def kernel(memory, x, y):
    mt = memory.T
    mtail = lax.slice(mt, (0, TAIL0), (DIM, M_ROWS))
    out_t, otail = _update(x, y, mt, mtail)
    out_t = lax.dynamic_update_slice(out_t, otail, (0, TAIL0))
    return out_t.T


# speedup vs baseline: 1.1800x; 1.0322x over previous
import dataclasses

import jax
import jax.numpy as jnp
from jax import lax
from jax.experimental import pallas as pl
from jax.experimental.pallas import tpu as pltpu
from jax.experimental.pallas import tpu_sc as plsc

M_ROWS = 1000000
DIM = 64
BATCH = 16384
MOM = 0.5

NC = 2
NS = 16
NW = NC * NS
L = 16
QUARTERS = DIM // L

BW = 512
OWN_W = 31232
NBLK = OWN_W // BW
NBLK_LAST = NBLK + 1
TAIL_W = M_ROWS - (NW - 1) * OWN_W - NBLK_LAST * BW
AUX_W = NBLK_LAST * BW + TAIL_W + L * 2
XS = BATCH // NS
RND = 2048
YB = 256
XCAP = 32


def _update_body(x_hbm, y_hbm, mt_hbm, mtail_hbm, out_hbm, otail_hbm,
                 ybuf_v, rows_c, pos_c, aux,
                 blk_a, blk_b, blk_t, wrk_col_a, wrk_pos_a, wrk_col_b,
                 wrk_pos_b, xbuf_a, xbuf_b,
                 sem_xa, sem_xb, sem_ia, sem_ib, sem_oa, sem_ob):
    cid = lax.axis_index("c")
    sid = lax.axis_index("s")
    wid = sid * NC + cid
    lo = wid * OWN_W
    hi = jnp.where(wid == NW - 1, jnp.int32(M_ROWS), lo + OWN_W)
    nblk = jnp.where(wid == NW - 1, jnp.int32(NBLK_LAST), jnp.int32(NBLK))
    lane = lax.iota(jnp.int32, L)


    neg1 = jnp.full((L,), -1, dtype=jnp.int32)

    @pl.loop(0, AUX_W // L)
    def _init(k):
        aux[pl.ds(k * L, L)] = neg1

    @pl.loop(0, BATCH // RND)
    def _round(r):
        base_r = r * RND
        pltpu.async_copy(y_hbm.at[pl.ds(base_r, RND)], ybuf_v, sem_ia).wait()

        def p1_body(cb, nacc):
            for t in range(YB // L):
                v = ybuf_v[pl.ds(cb * YB + t * L, L)]
                own = (v >= lo) & (v < hi)
                posv = lane + base_r + cb * YB + t * L
                plsc.store_compressed(rows_c.at[pl.ds(nacc, L)], v, mask=own)
                plsc.store_compressed(pos_c.at[pl.ds(nacc, L)], posv,
                                      mask=own)
                nacc = nacc + jnp.sum(own.astype(jnp.int32))
            return nacc

        n = lax.fori_loop(0, RND // YB, p1_body, jnp.int32(0))
        nblk2 = lax.div(n + (L - 1), jnp.int32(L))

        def p2_body(i, carry):
            base = i * L
            v = rows_c[pl.ds(base, L)] - lo
            p = pos_c[pl.ds(base, L)]
            valid = (lane + base) < n
            for j in range(L):
                plsc.store_scatter(aux, [v], p, mask=valid & (lane == j))
            return carry

        lax.fori_loop(0, nblk2, p2_body, jnp.int32(0))


    def make_copy(c0, blk, wrk_col, wrk_pos, xbuf, sem_i, sem_o, sem_x,
                  width, src=None, dst=None):
        def start_in():
            if src is not None:
                return pltpu.async_copy(src, blk, sem_i)
            return pltpu.async_copy(mt_hbm.at[:, pl.ds(c0, width)], blk,
                                    sem_i)

        def scan():
            def scan_piece(t, cnt):
                wv = aux[pl.ds(c0 - lo + t * L, L)]
                keep = wv >= jnp.int32(0)
                plsc.store_compressed(wrk_col.at[pl.ds(cnt, L)],
                                      lane + t * L, mask=keep)
                plsc.store_compressed(wrk_pos.at[pl.ds(cnt, L)], wv,
                                      mask=keep)
                return cnt + jnp.sum(keep.astype(jnp.int32))

            cnt = jnp.int32(0)
            for t in range(width // L):
                cnt = scan_piece(t, cnt)

            @pl.loop(0, jnp.minimum(cnt, XCAP))
            def _prefetch(u):
                pj = wrk_pos[pl.ds(u, L)][0]
                pltpu.async_copy(x_hbm.at[pj], xbuf.at[u], sem_x)

            return cnt

        def process(cnt):
            def chunk(w0):
                csz = jnp.minimum(cnt - w0, XCAP)

                @pl.when(w0 > 0)
                def _():
                    @pl.loop(0, csz)
                    def _prefetch(uu):
                        pj = wrk_pos[pl.ds(w0 + uu, L)][0]
                        pltpu.async_copy(x_hbm.at[pj], xbuf.at[uu], sem_x)

                @pl.loop(0, csz)
                def _drain(uu):
                    pj = wrk_pos[pl.ds(w0 + uu, L)][0]
                    pltpu.make_async_copy(x_hbm.at[pj], xbuf.at[uu],
                                          sem_x).wait()

                @pl.loop(0, csz)
                def _upd(uu):
                    u = w0 + uu
                    cj = wrk_col[pl.ds(u, L)][0]
                    colc = jnp.full((L,), cj, dtype=jnp.int32)
                    ws = [
                        plsc.load_gather(blk, [lane + q * L, colc]) * MOM
                        + xbuf[uu, pl.ds(q * L, L)] * (1.0 - MOM)
                        for q in range(QUARTERS)
                    ]
                    s = jnp.float32(0.0)
                    for wq in ws:
                        s = s + jnp.sum(wq * wq)
                    s = jnp.maximum(s, jnp.float32(1e-24))
                    sv = jnp.full((L,), s, dtype=jnp.float32)
                    bits = lax.bitcast_convert_type(sv, jnp.int32)
                    yv = lax.bitcast_convert_type(
                        jnp.int32(0x5F3759DF)
                        - lax.shift_right_logical(bits, 1),
                        jnp.float32,
                    )
                    for _ in range(3):
                        yv = yv * (1.5 - 0.5 * sv * yv * yv)
                    for q, wq in enumerate(ws):
                        plsc.store_scatter(blk, [lane + q * L, colc],
                                           wq * yv)

            pl.loop(0, cnt, step=XCAP)(chunk)

        def start_out():
            if dst is not None:
                return pltpu.async_copy(blk, dst, sem_o)
            return pltpu.async_copy(blk, out_hbm.at[:, pl.ds(c0, width)],
                                    sem_o)

        return start_in, scan, process, start_out

    def block_c0(k):
        return pl.multiple_of(lo + jnp.minimum(k, nblk - 1) * BW, BW)

    nblk_even = nblk + (nblk & 1)

    def pair_body(k):
        in_a, scan_a, proc_a, out_a = make_copy(
            block_c0(k), blk_a, wrk_col_a, wrk_pos_a, xbuf_a,
            sem_ia, sem_oa, sem_xa, BW)
        in_b, scan_b, proc_b, out_b = make_copy(
            block_c0(k + 1), blk_b, wrk_col_b, wrk_pos_b, xbuf_b,
            sem_ib, sem_ob, sem_xb, BW)
        @pl.when(k >= 2)
        def _():
            pltpu.make_async_copy(
                blk_a, out_hbm.at[:, pl.ds(block_c0(k - 2), BW)],
                sem_oa).wait()
            pltpu.make_async_copy(
                blk_b, out_hbm.at[:, pl.ds(block_c0(k - 1), BW)],
                sem_ob).wait()

        ca = in_a()
        cb = in_b()
        cnt_a = scan_a()
        cnt_b = scan_b()
        ca.wait()
        proc_a(cnt_a)
        out_a()
        cb.wait()
        proc_b(cnt_b)
        out_b()

    pl.loop(0, nblk_even, step=2)(pair_body)

    pltpu.make_async_copy(
        blk_a, out_hbm.at[:, pl.ds(block_c0(nblk_even - 2), BW)],
        sem_oa).wait()
    pltpu.make_async_copy(
        blk_b, out_hbm.at[:, pl.ds(block_c0(nblk_even - 1), BW)],
        sem_ob).wait()

    @pl.when(wid == NW - 1)
    def _tail():
        c0t = lo + NBLK_LAST * BW
        t_in, t_scan, t_proc, t_out = make_copy(
            c0t, blk_t, wrk_col_a, wrk_pos_a, xbuf_a,
            sem_ia, sem_oa, sem_xa, TAIL_W, src=mtail_hbm, dst=otail_hbm)
        ct = t_in()
        cnt_t = t_scan()
        ct.wait()
        t_proc(cnt_t)
        t_out().wait()


_cp = pltpu.CompilerParams()
if "needs_layout_passes" in pltpu.CompilerParams.__dataclass_fields__:
    _cp = dataclasses.replace(_cp, needs_layout_passes=False)

_update = pl.kernel(
    _update_body,
    out_type=(jax.ShapeDtypeStruct((DIM, M_ROWS), jnp.float32),
              jax.ShapeDtypeStruct((DIM, TAIL_W), jnp.float32)),
    mesh=plsc.VectorSubcoreMesh(
        core_axis_name="c", subcore_axis_name="s", num_cores=NC, num_subcores=NS
    ),
    compiler_params=_cp,
    scratch_types=[
        pltpu.VMEM((RND,), jnp.int32),
        pltpu.VMEM((RND + L,), jnp.int32),
        pltpu.VMEM((RND + L,), jnp.int32),
        pltpu.VMEM((AUX_W,), jnp.int32),
        pltpu.VMEM((DIM, BW), jnp.float32),
        pltpu.VMEM((DIM, BW), jnp.float32),
        pltpu.VMEM((DIM, TAIL_W), jnp.float32),
        pltpu.VMEM((BW + L,), jnp.int32),
        pltpu.VMEM((BW + L,), jnp.int32),
        pltpu.VMEM((BW + L,), jnp.int32),
        pltpu.VMEM((BW + L,), jnp.int32),
        pltpu.VMEM((XCAP, DIM), jnp.float32),
        pltpu.VMEM((XCAP, DIM), jnp.float32),
        pltpu.SemaphoreType.DMA,
        pltpu.SemaphoreType.DMA,
        pltpu.SemaphoreType.DMA,
        pltpu.SemaphoreType.DMA,
        pltpu.SemaphoreType.DMA,
        pltpu.SemaphoreType.DMA,
    ],
)


TAIL0 = (NW - 1) * OWN_W + NBLK_LAST * BW


def kernel(memory, x, y):
    mt = memory.T
    mtail = lax.slice(mt, (0, TAIL0), (DIM, M_ROWS))
    out_t, otail = _update(x, y, mt, mtail)
    out_t = lax.dynamic_update_slice(out_t, otail, (0, TAIL0))
    return out_t.T
